# baseline (speedup 1.0000x reference)
import functools

import jax
import jax.numpy as jnp
from jax import lax
from jax.experimental import pallas as pl
from jax.experimental.pallas import tpu as pltpu
from jax.experimental.pallas import tpu_sc as plsc

_N = 10000
_E = 320000
_F = 128
_H = 64
_C = 4

_CH = 128
_CPT = 80
_NBUF = 4
_NT = 32
_R = 10112
_EPAD = _NT * _CPT * _CH
_BR = 128
_NB = _R // _BR

_mesh = plsc.VectorSubcoreMesh(core_axis_name="c", subcore_axis_name="s")


@functools.partial(
    pl.kernel,
    out_type=jax.ShapeDtypeStruct((2, _R, 16), jnp.float32),
    mesh=_mesh,
    scratch_types=[
        pltpu.VMEM((_CPT, _CH), jnp.int32),
        pltpu.VMEM((_CH, 16), jnp.float32),
        pltpu.VMEM_SHARED((_R, 16), jnp.float32),
        pltpu.SemaphoreType.DMA,
    ],
)
def _sc_degree(dstp_hbm, ones_hbm, zeros_hbm, out_hbm, dst_v, ones_v, acc_sh,
               sem):
    c = lax.axis_index("c")
    s = lax.axis_index("s")
    gid = c * 16 + s
    pltpu.sync_copy(dstp_hbm.at[gid], dst_v)
    pltpu.sync_copy(ones_hbm, ones_v)

    @pl.when(s == 0)
    def _():
        pltpu.sync_copy(zeros_hbm, acc_sh)

    plsc.subcore_barrier()

    _K = 8

    def step(j, carry):
        pltpu.async_copy(ones_v, acc_sh.at[dst_v.at[j]], sem, add=True)

        @pl.when(j >= _K)
        def _():
            pltpu.make_async_copy(ones_v, acc_sh.at[dst_v.at[0]], sem).wait()

        return carry

    lax.fori_loop(0, _CPT, step, 0)

    def drain(j, carry):
        pltpu.make_async_copy(ones_v, acc_sh.at[dst_v.at[0]], sem).wait()
        return carry

    lax.fori_loop(0, _K, drain, 0)
    plsc.subcore_barrier()

    @pl.when(s == 0)
    def _():
        pltpu.sync_copy(acc_sh, out_hbm.at[c])


@functools.partial(
    pl.kernel,
    out_type=jax.ShapeDtypeStruct((2, _R, _H), jnp.float32),
    mesh=_mesh,
    compiler_params=pltpu.CompilerParams(use_tc_tiling_on_sc=False),
    scratch_types=[
        pltpu.VMEM((_CPT, _CH), jnp.int32),
        pltpu.VMEM((_CPT, _CH), jnp.int32),
        pltpu.VMEM((_NBUF, _CH, _H), jnp.float32),
        pltpu.VMEM_SHARED((_R, _H), jnp.float32),
        [pltpu.SemaphoreType.DMA] * _NBUF,
        [pltpu.SemaphoreType.DMA] * _NBUF,
    ],
)
def _sc_scatter(table_hbm, srcp_hbm, dstp_hbm, zeros_hbm, out_hbm,
                src_v, dst_v, rows_v, acc_sh, gsems, ssems):
    c = lax.axis_index("c")
    s = lax.axis_index("s")
    gid = c * 16 + s
    pltpu.sync_copy(srcp_hbm.at[gid], src_v)
    pltpu.sync_copy(dstp_hbm.at[gid], dst_v)

    @pl.when(s == 0)
    def _():
        pltpu.sync_copy(zeros_hbm, acc_sh)

    plsc.subcore_barrier()

    def fire_gather(j, b):
        pltpu.async_copy(table_hbm.at[src_v.at[j]], rows_v.at[b], gsems[b])

    for b in range(_NBUF):
        fire_gather(b, b)

    def outer(step, carry):
        for b in range(_NBUF):
            j = step * _NBUF + b
            pltpu.make_async_copy(
                table_hbm.at[src_v.at[j]], rows_v.at[b], gsems[b]).wait()
            pltpu.async_copy(rows_v.at[b], acc_sh.at[dst_v.at[j]], ssems[b],
                             add=True)
            pltpu.make_async_copy(
                rows_v.at[b], acc_sh.at[dst_v.at[j]], ssems[b]).wait()
            nj = j + _NBUF

            @pl.when(nj < _CPT)
            def _():
                fire_gather(nj, b)

        return carry

    lax.fori_loop(0, _CPT // _NBUF, outer, 0)
    plsc.subcore_barrier()

    @pl.when(s == 0)
    def _():
        pltpu.sync_copy(acc_sh, out_hbm.at[c])


def _norm_col(degp, j):
    deg = degp[0] + degp[1]
    degc = deg[:, 0:1] + 1.0
    rid = j * _BR + lax.broadcasted_iota(jnp.int32, (_BR, 1), 0)
    return jnp.where(rid < _N, lax.rsqrt(degc), 0.0)


def _tc_layer1_body(x_ref, degp_ref, w_ref, b_ref, g_ref):
    norm = _norm_col(degp_ref, pl.program_id(0))
    xw = jnp.dot(x_ref[...], w_ref[...], preferred_element_type=jnp.float32)
    g_ref[...] = (xw + b_ref[...]) * norm


def _tc_layer1(x_pad, degp, W1, b1):
    return pl.pallas_call(
        _tc_layer1_body,
        grid=(_NB,),
        in_specs=[
            pl.BlockSpec((_BR, _F), lambda j: (j, 0)),
            pl.BlockSpec((2, _BR, 16), lambda j: (0, j, 0)),
            pl.BlockSpec((_F, _H), lambda j: (0, 0)),
            pl.BlockSpec((1, _H), lambda j: (0, 0)),
        ],
        out_specs=pl.BlockSpec((_BR, _H), lambda j: (j, 0)),
        out_shape=jax.ShapeDtypeStruct((_R, _H), jnp.float32),
    )(x_pad, degp, W1, b1)


def _tc_layer2_body(acc_ref, g1_ref, degp_ref, w_ref, b_ref, g2_ref):
    norm = _norm_col(degp_ref, pl.program_id(0))
    h1 = jnp.maximum((acc_ref[0] + acc_ref[1] + g1_ref[...]) * norm, 0.0)
    hw = jnp.dot(h1, w_ref[...], preferred_element_type=jnp.float32)
    g2_ref[...] = (hw + b_ref[...]) * norm


def _tc_layer2(acc1, g1, degp, W2, b2):
    return pl.pallas_call(
        _tc_layer2_body,
        grid=(_NB,),
        in_specs=[
            pl.BlockSpec((2, _BR, _H), lambda j: (0, j, 0)),
            pl.BlockSpec((_BR, _H), lambda j: (j, 0)),
            pl.BlockSpec((2, _BR, 16), lambda j: (0, j, 0)),
            pl.BlockSpec((_H, _H), lambda j: (0, 0)),
            pl.BlockSpec((1, _H), lambda j: (0, 0)),
        ],
        out_specs=pl.BlockSpec((_BR, _H), lambda j: (j, 0)),
        out_shape=jax.ShapeDtypeStruct((_R, _H), jnp.float32),
    )(acc1, g1, degp, W2, b2)


def _tc_head_body(acc_ref, g2_ref, degp_ref, wd_ref, bd_ref, out_ref, psum):
    j = pl.program_id(0)
    norm = _norm_col(degp_ref, j)
    h2 = jnp.maximum((acc_ref[0] + acc_ref[1] + g2_ref[...]) * norm, 0.0)

    @pl.when(j == 0)
    def _():
        psum[...] = jnp.zeros_like(psum)

    psum[...] += jnp.sum(h2, axis=0, keepdims=True)

    @pl.when(j == _NB - 1)
    def _():
        logits = jnp.dot(psum[...], wd_ref[...],
                         preferred_element_type=jnp.float32) + bd_ref[...]
        m = jnp.max(logits, axis=-1, keepdims=True)
        e = jnp.exp(logits - m)
        out_ref[...] = e / jnp.sum(e, axis=-1, keepdims=True)


def _tc_head(acc2, g2, degp, Wd_pad, bd_pad):
    return pl.pallas_call(
        _tc_head_body,
        grid=(_NB,),
        in_specs=[
            pl.BlockSpec((2, _BR, _H), lambda j: (0, j, 0)),
            pl.BlockSpec((_BR, _H), lambda j: (j, 0)),
            pl.BlockSpec((2, _BR, 16), lambda j: (0, j, 0)),
            pl.BlockSpec((_H, 128), lambda j: (0, 0)),
            pl.BlockSpec((1, 128), lambda j: (0, 0)),
        ],
        out_specs=pl.BlockSpec((1, 128), lambda j: (0, 0)),
        out_shape=jax.ShapeDtypeStruct((1, 128), jnp.float32),
        scratch_shapes=[pltpu.VMEM((1, _H), jnp.float32)],
    )(acc2, g2, degp, Wd_pad, bd_pad)


def kernel(x, edge_index, W1, b1, W2, b2, Wd, bd):
    src = edge_index[0].astype(jnp.int32)
    dst = edge_index[1].astype(jnp.int32)
    pad_e = _EPAD - _E
    srcp = jnp.concatenate(
        [src, jnp.zeros((pad_e,), jnp.int32)]).reshape(_NT, _CPT, _CH)
    dstp = jnp.concatenate(
        [dst, jnp.full((pad_e,), _R - 1, jnp.int32)]).reshape(_NT, _CPT, _CH)
    x_pad = jnp.pad(x, ((0, _R - _N), (0, 0)))
    zeros64 = jnp.zeros((_R, _H), jnp.float32)
    zeros16 = jnp.zeros((_R, 16), jnp.float32)
    ones16 = jnp.ones((_CH, 16), jnp.float32)
    b1r = b1.reshape(1, _H)
    b2r = b2.reshape(1, _H)
    Wd_pad = jnp.zeros((_H, 128), jnp.float32).at[:, :_C].set(Wd)
    bd_pad = jnp.full((1, 128), -1e30, jnp.float32).at[0, :_C].set(bd)

    degp = _sc_degree(dstp, ones16, zeros16)
    g1 = _tc_layer1(x_pad, degp, W1, b1r)
    acc1 = _sc_scatter(g1, srcp, dstp, zeros64)
    g2 = _tc_layer2(acc1, g1, degp, W2, b2r)
    acc2 = _sc_scatter(g2, srcp, dstp, zeros64)
    out = _tc_head(acc2, g2, degp, Wd_pad, bd_pad)
    return out[:, :_C]


# speedup vs baseline: 2.0582x; 2.0582x over previous
import functools

import jax
import jax.numpy as jnp
from jax import lax
from jax.experimental import pallas as pl
from jax.experimental.pallas import tpu as pltpu
from jax.experimental.pallas import tpu_sc as plsc

_N = 10000
_E = 320000
_F = 128
_H = 64
_C = 4

_CH = 125
_CPT = 80
_NBUF = 4
_NT = 32
_R = 10112
_BR = 128
_NB = _R // _BR

_mesh = plsc.VectorSubcoreMesh(core_axis_name="c", subcore_axis_name="s")


@functools.partial(
    pl.kernel,
    out_type=jax.ShapeDtypeStruct((2, _R, 16), jnp.float32),
    mesh=_mesh,
    scratch_types=[
        pltpu.VMEM((_CPT, _CH), jnp.int32),
        pltpu.VMEM((_CH, 16), jnp.float32),
        pltpu.VMEM_SHARED((_R, 16), jnp.float32),
        pltpu.SemaphoreType.DMA,
    ],
)
def _sc_degree(dstp_hbm, ones_hbm, zeros_hbm, out_hbm, dst_v, ones_v, acc_sh,
               sem):
    c = lax.axis_index("c")
    s = lax.axis_index("s")
    gid = c * 16 + s
    pltpu.sync_copy(dstp_hbm.at[gid], dst_v)
    pltpu.sync_copy(ones_hbm, ones_v)

    @pl.when(s == 0)
    def _():
        pltpu.sync_copy(zeros_hbm, acc_sh)

    plsc.subcore_barrier()

    _K = 8

    def step(j, carry):
        pltpu.async_copy(ones_v, acc_sh.at[dst_v.at[j]], sem, add=True)

        @pl.when(j >= _K)
        def _():
            pltpu.make_async_copy(ones_v, acc_sh.at[dst_v.at[0]], sem).wait()

        return carry

    lax.fori_loop(0, _CPT, step, 0)

    def drain(j, carry):
        pltpu.make_async_copy(ones_v, acc_sh.at[dst_v.at[0]], sem).wait()
        return carry

    lax.fori_loop(0, _K, drain, 0)
    plsc.subcore_barrier()

    @pl.when(s == 0)
    def _():
        pltpu.sync_copy(acc_sh, out_hbm.at[c])


@functools.partial(
    pl.kernel,
    out_type=jax.ShapeDtypeStruct((2, _R, _H), jnp.float32),
    mesh=_mesh,
    compiler_params=pltpu.CompilerParams(use_tc_tiling_on_sc=False),
    scratch_types=[
        pltpu.VMEM((_CPT, _CH), jnp.int32),
        pltpu.VMEM((_CPT, _CH), jnp.int32),
        pltpu.VMEM((_NBUF, _CH, _H), jnp.float32),
        pltpu.VMEM_SHARED((_R, _H), jnp.float32),
        [pltpu.SemaphoreType.DMA] * _NBUF,
        [pltpu.SemaphoreType.DMA] * _NBUF,
    ],
)
def _sc_scatter(table_hbm, srcp_hbm, dstp_hbm, zeros_hbm, out_hbm,
                src_v, dst_v, rows_v, acc_sh, gsems, ssems):
    c = lax.axis_index("c")
    s = lax.axis_index("s")
    gid = c * 16 + s
    pltpu.sync_copy(srcp_hbm.at[gid], src_v)
    pltpu.sync_copy(dstp_hbm.at[gid], dst_v)

    @pl.when(s == 0)
    def _():
        pltpu.sync_copy(zeros_hbm, acc_sh)

    plsc.subcore_barrier()

    def fire_gather(j, b):
        pltpu.async_copy(table_hbm.at[src_v.at[j]], rows_v.at[b], gsems[b])

    for b in range(_NBUF):
        fire_gather(b, b)

    def outer(step, carry):
        for b in range(_NBUF):
            j = step * _NBUF + b
            pltpu.make_async_copy(
                table_hbm.at[src_v.at[j]], rows_v.at[b], gsems[b]).wait()
            pltpu.async_copy(rows_v.at[b], acc_sh.at[dst_v.at[j]], ssems[b],
                             add=True)
            pltpu.make_async_copy(
                rows_v.at[b], acc_sh.at[dst_v.at[j]], ssems[b]).wait()
            nj = j + _NBUF

            @pl.when(nj < _CPT)
            def _():
                fire_gather(nj, b)

        return carry

    lax.fori_loop(0, _CPT // _NBUF, outer, 0)
    plsc.subcore_barrier()

    @pl.when(s == 0)
    def _():
        pltpu.sync_copy(acc_sh, out_hbm.at[c])


def _norm_col(degp, j):
    deg = degp[0] + degp[1]
    degc = deg[:, 0:1] + 1.0
    rid = j * _BR + lax.broadcasted_iota(jnp.int32, (_BR, 1), 0)
    return jnp.where(rid < _N, lax.rsqrt(degc), 0.0)


def _tc_layer1_body(x_ref, degp_ref, w_ref, b_ref, g_ref):
    norm = _norm_col(degp_ref, pl.program_id(0))
    xw = jnp.dot(x_ref[...], w_ref[...], preferred_element_type=jnp.float32)
    g_ref[...] = (xw + b_ref[...]) * norm


def _tc_layer1(x_pad, degp, W1, b1):
    return pl.pallas_call(
        _tc_layer1_body,
        grid=(_NB,),
        in_specs=[
            pl.BlockSpec((_BR, _F), lambda j: (j, 0)),
            pl.BlockSpec((2, _BR, 16), lambda j: (0, j, 0)),
            pl.BlockSpec((_F, _H), lambda j: (0, 0)),
            pl.BlockSpec((1, _H), lambda j: (0, 0)),
        ],
        out_specs=pl.BlockSpec((_BR, _H), lambda j: (j, 0)),
        out_shape=jax.ShapeDtypeStruct((_R, _H), jnp.float32),
    )(x_pad, degp, W1, b1)


def _tc_layer2_body(acc_ref, g1_ref, degp_ref, w_ref, b_ref, g2_ref):
    norm = _norm_col(degp_ref, pl.program_id(0))
    h1 = jnp.maximum((acc_ref[0] + acc_ref[1] + g1_ref[...]) * norm, 0.0)
    hw = jnp.dot(h1, w_ref[...], preferred_element_type=jnp.float32)
    g2_ref[...] = (hw + b_ref[...]) * norm


def _tc_layer2(acc1, g1, degp, W2, b2):
    return pl.pallas_call(
        _tc_layer2_body,
        grid=(_NB,),
        in_specs=[
            pl.BlockSpec((2, _BR, _H), lambda j: (0, j, 0)),
            pl.BlockSpec((_BR, _H), lambda j: (j, 0)),
            pl.BlockSpec((2, _BR, 16), lambda j: (0, j, 0)),
            pl.BlockSpec((_H, _H), lambda j: (0, 0)),
            pl.BlockSpec((1, _H), lambda j: (0, 0)),
        ],
        out_specs=pl.BlockSpec((_BR, _H), lambda j: (j, 0)),
        out_shape=jax.ShapeDtypeStruct((_R, _H), jnp.float32),
    )(acc1, g1, degp, W2, b2)


def _tc_head_body(acc_ref, g2_ref, degp_ref, wd_ref, bd_ref, out_ref, psum):
    j = pl.program_id(0)
    norm = _norm_col(degp_ref, j)
    h2 = jnp.maximum((acc_ref[0] + acc_ref[1] + g2_ref[...]) * norm, 0.0)

    @pl.when(j == 0)
    def _():
        psum[...] = jnp.zeros_like(psum)

    psum[...] += jnp.sum(h2, axis=0, keepdims=True)

    @pl.when(j == _NB - 1)
    def _():
        logits = jnp.dot(psum[...], wd_ref[...],
                         preferred_element_type=jnp.float32) + bd_ref[...]
        m = jnp.max(logits, axis=-1, keepdims=True)
        e = jnp.exp(logits - m)
        out_ref[...] = e / jnp.sum(e, axis=-1, keepdims=True)


def _tc_head(acc2, g2, degp, Wd_pad, bd_pad):
    return pl.pallas_call(
        _tc_head_body,
        grid=(_NB,),
        in_specs=[
            pl.BlockSpec((2, _BR, _H), lambda j: (0, j, 0)),
            pl.BlockSpec((_BR, _H), lambda j: (j, 0)),
            pl.BlockSpec((2, _BR, 16), lambda j: (0, j, 0)),
            pl.BlockSpec((_H, 128), lambda j: (0, 0)),
            pl.BlockSpec((1, 128), lambda j: (0, 0)),
        ],
        out_specs=pl.BlockSpec((1, 128), lambda j: (0, 0)),
        out_shape=jax.ShapeDtypeStruct((1, 128), jnp.float32),
        scratch_shapes=[pltpu.VMEM((1, _H), jnp.float32)],
    )(acc2, g2, degp, Wd_pad, bd_pad)


def kernel(x, edge_index, W1, b1, W2, b2, Wd, bd):
    srcp = edge_index[0].astype(jnp.int32).reshape(_NT, _CPT, _CH)
    dstp = edge_index[1].astype(jnp.int32).reshape(_NT, _CPT, _CH)
    x_pad = jnp.pad(x, ((0, _R - _N), (0, 0)))
    zeros64 = jnp.zeros((_R, _H), jnp.float32)
    zeros16 = jnp.zeros((_R, 16), jnp.float32)
    ones16 = jnp.ones((_CH, 16), jnp.float32)
    b1r = b1.reshape(1, _H)
    b2r = b2.reshape(1, _H)
    Wd_pad = jnp.zeros((_H, 128), jnp.float32).at[:, :_C].set(Wd)
    bd_pad = jnp.full((1, 128), -1e30, jnp.float32).at[0, :_C].set(bd)

    degp = _sc_degree(dstp, ones16, zeros16)
    g1 = _tc_layer1(x_pad, degp, W1, b1r)
    acc1 = _sc_scatter(g1, srcp, dstp, zeros64)
    g2 = _tc_layer2(acc1, g1, degp, W2, b2r)
    acc2 = _sc_scatter(g2, srcp, dstp, zeros64)
    out = _tc_head(acc2, g2, degp, Wd_pad, bd_pad)
    return out[:, :_C]


# speedup vs baseline: 3.1913x; 1.5505x over previous
import functools

import jax
import jax.numpy as jnp
from jax import lax
from jax.experimental import pallas as pl
from jax.experimental.pallas import tpu as pltpu
from jax.experimental.pallas import tpu_sc as plsc

_N = 10000
_E = 320000
_F = 128
_H = 64
_C = 4

_CH = 125
_CPT = 80
_NBUF = 4
_NT = 32
_R = 10112
_BR = 128
_NB = _R // _BR

_mesh = plsc.VectorSubcoreMesh(core_axis_name="c", subcore_axis_name="s")


@functools.partial(
    pl.kernel,
    out_type=jax.ShapeDtypeStruct((2, _R, 16), jnp.float32),
    mesh=_mesh,
    scratch_types=[
        pltpu.VMEM((_CPT, _CH), jnp.int32),
        pltpu.VMEM((_CH, 16), jnp.float32),
        pltpu.VMEM_SHARED((_R, 16), jnp.float32),
        pltpu.SemaphoreType.DMA,
    ],
)
def _sc_degree(ei_hbm, ones_hbm, zeros_hbm, out_hbm, dst_v, ones_v, acc_sh,
               sem):
    c = lax.axis_index("c")
    s = lax.axis_index("s")
    gid = c * 16 + s
    pltpu.sync_copy(ei_hbm.at[1].at[gid], dst_v)
    pltpu.sync_copy(ones_hbm, ones_v)

    @pl.when(s == 0)
    def _():
        pltpu.sync_copy(zeros_hbm, acc_sh)

    plsc.subcore_barrier()

    _K = 8

    def step(j, carry):
        pltpu.async_copy(ones_v, acc_sh.at[dst_v.at[j]], sem, add=True)

        @pl.when(j >= _K)
        def _():
            pltpu.make_async_copy(ones_v, acc_sh.at[dst_v.at[0]], sem).wait()

        return carry

    lax.fori_loop(0, _CPT, step, 0)

    def drain(j, carry):
        pltpu.make_async_copy(ones_v, acc_sh.at[dst_v.at[0]], sem).wait()
        return carry

    lax.fori_loop(0, _K, drain, 0)
    plsc.subcore_barrier()

    @pl.when(s == 0)
    def _():
        pltpu.sync_copy(acc_sh, out_hbm.at[c])


@functools.partial(
    pl.kernel,
    out_type=jax.ShapeDtypeStruct((2, _R, _H), jnp.float32),
    mesh=_mesh,
    compiler_params=pltpu.CompilerParams(use_tc_tiling_on_sc=False),
    scratch_types=[
        pltpu.VMEM((_CPT, _CH), jnp.int32),
        pltpu.VMEM((_CPT, _CH), jnp.int32),
        pltpu.VMEM((_NBUF, _CH, _H), jnp.float32),
        pltpu.VMEM_SHARED((_R, _H), jnp.float32),
        [pltpu.SemaphoreType.DMA] * _NBUF,
        [pltpu.SemaphoreType.DMA] * _NBUF,
    ],
)
def _sc_scatter(table_hbm, ei_hbm, zeros_hbm, out_hbm,
                src_v, dst_v, rows_v, acc_sh, gsems, ssems):
    c = lax.axis_index("c")
    s = lax.axis_index("s")
    gid = c * 16 + s
    pltpu.sync_copy(ei_hbm.at[0].at[gid], src_v)
    pltpu.sync_copy(ei_hbm.at[1].at[gid], dst_v)

    @pl.when(s == 0)
    def _():
        pltpu.sync_copy(zeros_hbm, acc_sh)

    plsc.subcore_barrier()

    def fire_gather(j, b):
        pltpu.async_copy(table_hbm.at[src_v.at[j]], rows_v.at[b], gsems[b])

    for b in range(_NBUF):
        fire_gather(b, b)

    def outer(step, carry):
        for b in range(_NBUF):
            j = step * _NBUF + b
            pltpu.make_async_copy(
                table_hbm.at[src_v.at[j]], rows_v.at[b], gsems[b]).wait()
            pltpu.async_copy(rows_v.at[b], acc_sh.at[dst_v.at[j]], ssems[b],
                             add=True)
            pltpu.make_async_copy(
                rows_v.at[b], acc_sh.at[dst_v.at[j]], ssems[b]).wait()
            nj = j + _NBUF

            @pl.when(nj < _CPT)
            def _():
                fire_gather(nj, b)

        return carry

    lax.fori_loop(0, _CPT // _NBUF, outer, 0)
    plsc.subcore_barrier()

    @pl.when(s == 0)
    def _():
        pltpu.sync_copy(acc_sh, out_hbm.at[c])


def _norm_col(degp):
    deg = degp[0] + degp[1]
    degc = deg[:, 0:1] + 1.0
    rid = lax.broadcasted_iota(jnp.int32, (_R, 1), 0)
    return jnp.where(rid < _N, lax.rsqrt(degc), 0.0)


def _tc_layer1_body(x_ref, degp_ref, w_ref, b_ref, g_ref):
    norm = _norm_col(degp_ref)
    xw = jnp.dot(x_ref[...], w_ref[...], preferred_element_type=jnp.float32)
    g_ref[...] = (xw + b_ref[...]) * norm


def _tc_layer1(x_pad, degp, W1, b1):
    return pl.pallas_call(
        _tc_layer1_body,
        out_shape=jax.ShapeDtypeStruct((_R, _H), jnp.float32),
    )(x_pad, degp, W1, b1)


def _tc_layer2_body(acc_ref, g1_ref, degp_ref, w_ref, b_ref, g2_ref):
    norm = _norm_col(degp_ref)
    h1 = jnp.maximum((acc_ref[0] + acc_ref[1] + g1_ref[...]) * norm, 0.0)
    hw = jnp.dot(h1, w_ref[...], preferred_element_type=jnp.float32)
    g2_ref[...] = (hw + b_ref[...]) * norm


def _tc_layer2(acc1, g1, degp, W2, b2):
    return pl.pallas_call(
        _tc_layer2_body,
        out_shape=jax.ShapeDtypeStruct((_R, _H), jnp.float32),
    )(acc1, g1, degp, W2, b2)


def _tc_head_body(acc_ref, g2_ref, degp_ref, wd_ref, bd_ref, out_ref):
    norm = _norm_col(degp_ref)
    h2 = jnp.maximum((acc_ref[0] + acc_ref[1] + g2_ref[...]) * norm, 0.0)
    pooled = jnp.sum(h2, axis=0, keepdims=True)
    logits = jnp.dot(pooled, wd_ref[...],
                     preferred_element_type=jnp.float32) + bd_ref[...]
    m = jnp.max(logits, axis=-1, keepdims=True)
    e = jnp.exp(logits - m)
    out_ref[...] = e / jnp.sum(e, axis=-1, keepdims=True)


def _tc_head(acc2, g2, degp, Wd_pad, bd_pad):
    return pl.pallas_call(
        _tc_head_body,
        out_shape=jax.ShapeDtypeStruct((1, 128), jnp.float32),
    )(acc2, g2, degp, Wd_pad, bd_pad)


def kernel(x, edge_index, W1, b1, W2, b2, Wd, bd):
    ei = edge_index.astype(jnp.int32).reshape(2, _NT, _CPT, _CH)
    x_pad = jnp.pad(x, ((0, _R - _N), (0, 0)))
    zeros64 = jnp.zeros((_R, _H), jnp.float32)
    zeros16 = jnp.zeros((_R, 16), jnp.float32)
    ones16 = jnp.ones((_CH, 16), jnp.float32)
    b1r = b1.reshape(1, _H)
    b2r = b2.reshape(1, _H)
    Wd_pad = jnp.zeros((_H, 128), jnp.float32).at[:, :_C].set(Wd)
    bd_pad = jnp.full((1, 128), -1e30, jnp.float32).at[0, :_C].set(bd)

    degp = _sc_degree(ei, ones16, zeros16)
    g1 = _tc_layer1(x_pad, degp, W1, b1r)
    acc1 = _sc_scatter(g1, ei, zeros64)
    g2 = _tc_layer2(acc1, g1, degp, W2, b2r)
    acc2 = _sc_scatter(g2, ei, zeros64)
    out = _tc_head(acc2, g2, degp, Wd_pad, bd_pad)
    return out[:, :_C]


# speedup vs baseline: 3.7742x; 1.1826x over previous
import functools

import jax
import jax.numpy as jnp
from jax import lax
from jax.experimental import pallas as pl
from jax.experimental.pallas import tpu as pltpu
from jax.experimental.pallas import tpu_sc as plsc

_N = 10000
_E = 320000
_F = 128
_H = 64
_C = 4

_CH = 125
_CPT = 80
_NBUF = 4
_NT = 32
_R = 10112
_RP = _R // 2
_DW = 8

_mesh = plsc.VectorSubcoreMesh(core_axis_name="c", subcore_axis_name="s")


@functools.partial(
    pl.kernel,
    out_type=jax.ShapeDtypeStruct((2, _R, _DW), jnp.float32),
    mesh=_mesh,
    scratch_types=[
        pltpu.VMEM((_CPT, _CH), jnp.int32),
        pltpu.VMEM((_CH, _DW), jnp.float32),
        pltpu.VMEM_SHARED((_R, _DW), jnp.float32),
        pltpu.SemaphoreType.DMA,
    ],
)
def _sc_degree(ei_hbm, ones_hbm, zeros_hbm, out_hbm, dst_v, ones_v, acc_sh,
               sem):
    c = lax.axis_index("c")
    s = lax.axis_index("s")
    gid = c * 16 + s
    pltpu.sync_copy(ei_hbm.at[1].at[gid], dst_v)
    pltpu.sync_copy(ones_hbm, ones_v)

    @pl.when(s == 0)
    def _():
        pltpu.sync_copy(zeros_hbm, acc_sh)

    plsc.subcore_barrier()

    _K = 8

    def step(j, carry):
        pltpu.async_copy(ones_v, acc_sh.at[dst_v.at[j]], sem, add=True)

        @pl.when(j >= _K)
        def _():
            pltpu.make_async_copy(ones_v, acc_sh.at[dst_v.at[0]], sem).wait()

        return carry

    lax.fori_loop(0, _CPT, step, 0)

    def drain(j, carry):
        pltpu.make_async_copy(ones_v, acc_sh.at[dst_v.at[0]], sem).wait()
        return carry

    lax.fori_loop(0, _K, drain, 0)
    plsc.subcore_barrier()

    @pl.when(s == 0)
    def _():
        pltpu.sync_copy(acc_sh, out_hbm.at[c])


@functools.partial(
    pl.kernel,
    out_type=jax.ShapeDtypeStruct((2, _R, _H), jnp.float32),
    mesh=_mesh,
    compiler_params=pltpu.CompilerParams(use_tc_tiling_on_sc=False),
    scratch_types=[
        pltpu.VMEM((_CPT, _CH), jnp.int32),
        pltpu.VMEM((_CPT, _CH), jnp.int32),
        pltpu.VMEM((_NBUF, _CH, _H), jnp.float32),
        pltpu.VMEM_SHARED((_R, _H), jnp.float32),
        [pltpu.SemaphoreType.DMA] * _NBUF,
        [pltpu.SemaphoreType.DMA] * _NBUF,
    ],
)
def _sc_scatter(table_hbm, ei_hbm, zeros_hbm, out_hbm,
                src_v, dst_v, rows_v, acc_sh, gsems, ssems):
    c = lax.axis_index("c")
    s = lax.axis_index("s")
    gid = c * 16 + s
    pltpu.sync_copy(ei_hbm.at[0].at[gid], src_v)
    pltpu.sync_copy(ei_hbm.at[1].at[gid], dst_v)

    @pl.when(s == 0)
    def _():
        pltpu.sync_copy(zeros_hbm, acc_sh)

    plsc.subcore_barrier()

    def fire_gather(j, b):
        pltpu.async_copy(table_hbm.at[src_v.at[j]], rows_v.at[b], gsems[b])

    for b in range(_NBUF):
        fire_gather(b, b)

    def outer(step, carry):
        for b in range(_NBUF):
            j = step * _NBUF + b
            pltpu.make_async_copy(
                table_hbm.at[src_v.at[j]], rows_v.at[b], gsems[b]).wait()
            pltpu.async_copy(rows_v.at[b], acc_sh.at[dst_v.at[j]], ssems[b],
                             add=True)
            pltpu.make_async_copy(
                rows_v.at[b], acc_sh.at[dst_v.at[j]], ssems[b]).wait()
            nj = j + _NBUF

            @pl.when(nj < _CPT)
            def _():
                fire_gather(nj, b)

        return carry

    lax.fori_loop(0, _CPT // _NBUF, outer, 0)
    plsc.subcore_barrier()

    @pl.when(s == 0)
    def _():
        pltpu.sync_copy(acc_sh, out_hbm.at[c])


def _norm_packed(degt):
    d = degt[0] + degt[1]
    i0 = lax.broadcasted_iota(jnp.int32, (16, 2), 0)
    i1 = lax.broadcasted_iota(jnp.int32, (16, 2), 1)
    sel = (i0 == i1 * _DW).astype(jnp.float32)
    deg2 = jnp.dot(d, sel, preferred_element_type=jnp.float32) + 1.0
    r0 = lax.broadcasted_iota(jnp.int32, (_RP, 2), 0)
    r1 = lax.broadcasted_iota(jnp.int32, (_RP, 2), 1)
    node = 2 * r0 + r1
    norm2 = jnp.where(node < _N, lax.rsqrt(deg2), 0.0)
    j0 = lax.broadcasted_iota(jnp.int32, (2, 128), 0)
    j1 = lax.broadcasted_iota(jnp.int32, (2, 128), 1)
    rep = (j0 == j1 // _H).astype(jnp.float32)
    return jnp.dot(norm2, rep, preferred_element_type=jnp.float32)


def _tc_matmul1_body(x_ref, w_ref, b_ref, t_ref):
    t_ref[...] = jnp.dot(x_ref[...], w_ref[...],
                         preferred_element_type=jnp.float32) + b_ref[...]


def _tc_matmul1(x_packed, W1bd, b1p):
    return pl.pallas_call(
        _tc_matmul1_body,
        out_shape=jax.ShapeDtypeStruct((_RP, 128), jnp.float32),
    )(x_packed, W1bd, b1p)


def _tc_scale_body(t_ref, degt_ref, g_ref):
    g_ref[...] = t_ref[...] * _norm_packed(degt_ref)


def _tc_scale(t1, degt):
    return pl.pallas_call(
        _tc_scale_body,
        out_shape=jax.ShapeDtypeStruct((_RP, 128), jnp.float32),
    )(t1, degt)


def _tc_layer2_body(acc_ref, g1_ref, degt_ref, w_ref, b_ref, g2_ref):
    np_ = _norm_packed(degt_ref)
    h1 = jnp.maximum((acc_ref[0] + acc_ref[1] + g1_ref[...]) * np_, 0.0)
    hw = jnp.dot(h1, w_ref[...], preferred_element_type=jnp.float32)
    g2_ref[...] = (hw + b_ref[...]) * np_


def _tc_layer2(acc1p, g1p, degt, W2bd, b2p):
    return pl.pallas_call(
        _tc_layer2_body,
        out_shape=jax.ShapeDtypeStruct((_RP, 128), jnp.float32),
    )(acc1p, g1p, degt, W2bd, b2p)


def _tc_head_body(acc_ref, g2_ref, degt_ref, wd_ref, bd_ref, out_ref):
    np_ = _norm_packed(degt_ref)
    h2 = jnp.maximum((acc_ref[0] + acc_ref[1] + g2_ref[...]) * np_, 0.0)
    pooled = jnp.sum(h2, axis=0, keepdims=True)
    logits = jnp.dot(pooled, wd_ref[...],
                     preferred_element_type=jnp.float32) + bd_ref[...]
    m = jnp.max(logits, axis=-1, keepdims=True)
    e = jnp.exp(logits - m)
    out_ref[...] = e / jnp.sum(e, axis=-1, keepdims=True)


def _tc_head(acc2p, g2p, degt, Wd_fold, bd_pad):
    return pl.pallas_call(
        _tc_head_body,
        out_shape=jax.ShapeDtypeStruct((1, 128), jnp.float32),
    )(acc2p, g2p, degt, Wd_fold, bd_pad)


def kernel(x, edge_index, W1, b1, W2, b2, Wd, bd):
    ei = edge_index.astype(jnp.int32).reshape(2, _NT, _CPT, _CH)
    x_packed = jnp.pad(x, ((0, _R - _N), (0, 0))).reshape(_RP, 2 * _F)
    zeros64 = jnp.zeros((_R, _H), jnp.float32)
    zeros8 = jnp.zeros((_R, _DW), jnp.float32)
    ones8 = jnp.ones((_CH, _DW), jnp.float32)
    W1bd = jnp.zeros((2 * _F, 128), jnp.float32)
    W1bd = W1bd.at[:_F, :_H].set(W1).at[_F:, _H:].set(W1)
    W2bd = jnp.zeros((128, 128), jnp.float32)
    W2bd = W2bd.at[:_H, :_H].set(W2).at[_H:, _H:].set(W2)
    b1p = jnp.concatenate([b1, b1]).reshape(1, 128)
    b2p = jnp.concatenate([b2, b2]).reshape(1, 128)
    Wd_pad = jnp.zeros((_H, 128), jnp.float32).at[:, :_C].set(Wd)
    Wd_fold = jnp.concatenate([Wd_pad, Wd_pad], axis=0)
    bd_pad = jnp.full((1, 128), -1e30, jnp.float32).at[0, :_C].set(bd)

    degp = _sc_degree(ei, ones8, zeros8)
    t1 = _tc_matmul1(x_packed, W1bd, b1p)
    degt = degp.reshape(2, _RP, 2 * _DW)
    g1p = _tc_scale(t1, degt)
    g1 = g1p.reshape(_R, _H)
    acc1 = _sc_scatter(g1, ei, zeros64)
    acc1p = acc1.reshape(2, _RP, 128)
    g2p = _tc_layer2(acc1p, g1p, degt, W2bd, b2p)
    g2 = g2p.reshape(_R, _H)
    acc2 = _sc_scatter(g2, ei, zeros64)
    acc2p = acc2.reshape(2, _RP, 128)
    out = _tc_head(acc2p, g2p, degt, Wd_fold, bd_pad)
    return out[:, :_C]
